# baseline (speedup 1.0000x reference)
import jax
import jax.numpy as jnp
from jax import lax
from jax.experimental import pallas as pl
from jax.experimental.pallas import tpu as pltpu
from jax.experimental.pallas import tpu_sc as plsc

NUM_CLASSES = 80
IOU_THRESHOLD = 0.5
M = 100
N = 20000
LANES = 16
NW = 32
PPW = 640
VPW = PPW // LANES


def _roi_body(px1_hbm, py1_hbm, px2_hbm, py2_hbm, gt_hbm, combo_hbm,
              vals_out, idxs_out, labs_out, cls_out,
              px1_v, py1_v, px2_v, py2_v, gt_v, ga_v, combo_v,
              vals_v, idxs_v, labs_v, clso_v):
    c = lax.axis_index("c")
    s = lax.axis_index("s")
    wid = s * 2 + c
    base = jnp.minimum(wid * PPW, N - PPW)

    pltpu.sync_copy(px1_hbm.at[pl.ds(base, PPW)], px1_v)
    pltpu.sync_copy(py1_hbm.at[pl.ds(base, PPW)], py1_v)
    pltpu.sync_copy(px2_hbm.at[pl.ds(base, PPW)], px2_v)
    pltpu.sync_copy(py2_hbm.at[pl.ds(base, PPW)], py2_v)
    pltpu.sync_copy(gt_hbm, gt_v)
    pltpu.sync_copy(combo_hbm, combo_v)

    def ga_body(g, carry):
        gx1 = gt_v[0, g, :]
        gy1 = gt_v[1, g, :]
        gx2 = gt_v[2, g, :]
        gy2 = gt_v[3, g, :]
        ga_v[g, :] = (gx2 - gx1) * (gy2 - gy1)
        return carry

    lax.fori_loop(0, M, ga_body, 0)

    def j_body(j, carry):
        o = j * LANES
        px1 = px1_v[pl.ds(o, LANES)]
        py1 = py1_v[pl.ds(o, LANES)]
        px2 = px2_v[pl.ds(o, LANES)]
        py2 = py2_v[pl.ds(o, LANES)]
        parea = (px2 - px1) * (py2 - py1)

        def g_body(g, st):
            bval, bcombo = st
            gx1 = gt_v[0, g, :]
            gy1 = gt_v[1, g, :]
            gx2 = gt_v[2, g, :]
            gy2 = gt_v[3, g, :]
            ga = ga_v[g, :]
            combo = combo_v[g, :]
            w = jnp.maximum(jnp.minimum(px2, gx2) - jnp.maximum(px1, gx1), 0.0)
            h = jnp.minimum(py2, gy2) - jnp.maximum(py1, gy1)
            inter = w * h
            iou = inter / (parea + ga - inter)
            upd = iou > bval
            bval = jnp.where(upd, iou, bval)
            bcombo = jnp.where(upd, combo, bcombo)
            return bval, bcombo

        init = (jnp.zeros((LANES,), jnp.float32), combo_v[0, :])
        bval, bcombo = lax.fori_loop(0, M, g_body, init)

        fg = bval >= IOU_THRESHOLD
        zero_i = jnp.zeros((LANES,), jnp.int32)
        bidx = lax.shift_right_logical(bcombo, 7)
        cls = lax.bitwise_and(bcombo, zero_i + 127)
        vals_v[pl.ds(o, LANES)] = bval
        idxs_v[pl.ds(o, LANES)] = bidx
        labs_v[pl.ds(o, LANES)] = jnp.where(fg, zero_i + 1, zero_i)
        clso_v[pl.ds(o, LANES)] = jnp.where(fg, cls, zero_i + NUM_CLASSES)
        return carry

    lax.fori_loop(0, VPW, j_body, 0)

    pltpu.sync_copy(vals_v, vals_out.at[pl.ds(base, PPW)])
    pltpu.sync_copy(idxs_v, idxs_out.at[pl.ds(base, PPW)])
    pltpu.sync_copy(labs_v, labs_out.at[pl.ds(base, PPW)])
    pltpu.sync_copy(clso_v, cls_out.at[pl.ds(base, PPW)])


_roi = pl.kernel(
    _roi_body,
    out_type=(jax.ShapeDtypeStruct((N,), jnp.float32),
              jax.ShapeDtypeStruct((N,), jnp.int32),
              jax.ShapeDtypeStruct((N,), jnp.int32),
              jax.ShapeDtypeStruct((N,), jnp.int32)),
    mesh=plsc.VectorSubcoreMesh(core_axis_name="c", subcore_axis_name="s"),
    scratch_types=[
        pltpu.VMEM((PPW,), jnp.float32),
        pltpu.VMEM((PPW,), jnp.float32),
        pltpu.VMEM((PPW,), jnp.float32),
        pltpu.VMEM((PPW,), jnp.float32),
        pltpu.VMEM((4, M, LANES), jnp.float32),
        pltpu.VMEM((M, LANES), jnp.float32),
        pltpu.VMEM((M, LANES), jnp.int32),
        pltpu.VMEM((PPW,), jnp.float32),
        pltpu.VMEM((PPW,), jnp.int32),
        pltpu.VMEM((PPW,), jnp.int32),
        pltpu.VMEM((PPW,), jnp.int32),
    ],
)


def kernel(proposal_boxes, gt_boxes, gt_classes):
    pb = proposal_boxes.astype(jnp.float32)
    gt_rep = jnp.broadcast_to(
        jnp.transpose(gt_boxes.astype(jnp.float32))[:, :, None], (4, M, LANES))
    combo = (jnp.arange(M, dtype=jnp.int32) * 128
             + gt_classes.astype(jnp.int32))
    combo_rep = jnp.broadcast_to(combo[:, None], (M, LANES))
    return _roi(pb[:, 0], pb[:, 1], pb[:, 2], pb[:, 3], gt_rep, combo_rep)


# speedup vs baseline: 1.0017x; 1.0017x over previous
import jax
import jax.numpy as jnp
from jax import lax
from jax.experimental import pallas as pl
from jax.experimental.pallas import tpu as pltpu
from jax.experimental.pallas import tpu_sc as plsc

NUM_CLASSES = 80
IOU_THRESHOLD = 0.5
M = 100
N = 20000
LANES = 16
NW = 32
PPW = 640
VPW = PPW // LANES
JB = 2


def _roi_body(pb_hbm, gt_hbm, combo_hbm,
              vals_out, idxs_out, labs_out, cls_out,
              px1_v, py1_v, px2_v, py2_v, gt_v, combo_v,
              vals_v, idxs_v, labs_v, clso_v):
    c = lax.axis_index("c")
    s = lax.axis_index("s")
    wid = s * 2 + c
    base = jnp.minimum(wid * PPW, N - PPW)

    pltpu.sync_copy(pb_hbm.at[pl.ds(base, PPW)], px1_v)
    pltpu.sync_copy(pb_hbm.at[pl.ds(N + base, PPW)], py1_v)
    pltpu.sync_copy(pb_hbm.at[pl.ds(2 * N + base, PPW)], px2_v)
    pltpu.sync_copy(pb_hbm.at[pl.ds(3 * N + base, PPW)], py2_v)
    pltpu.sync_copy(gt_hbm, gt_v)
    pltpu.sync_copy(combo_hbm, combo_v)

    def j_body(j, carry):
        o = j * (LANES * JB)

        px1 = [px1_v[pl.ds(o + k * LANES, LANES)] for k in range(JB)]
        py1 = [py1_v[pl.ds(o + k * LANES, LANES)] for k in range(JB)]
        px2 = [px2_v[pl.ds(o + k * LANES, LANES)] for k in range(JB)]
        py2 = [py2_v[pl.ds(o + k * LANES, LANES)] for k in range(JB)]
        parea = [(px2[k] - px1[k]) * (py2[k] - py1[k]) for k in range(JB)]

        def g_body(g, st):
            bval, bcombo = st
            gx1 = gt_v[0, g, :]
            gy1 = gt_v[1, g, :]
            gx2 = gt_v[2, g, :]
            gy2 = gt_v[3, g, :]
            ga = gt_v[4, g, :]
            combo = combo_v[g, :]
            nbval = []
            nbcombo = []
            for k in range(JB):
                w = jnp.maximum(
                    jnp.minimum(px2[k], gx2) - jnp.maximum(px1[k], gx1), 0.0)
                h = jnp.minimum(py2[k], gy2) - jnp.maximum(py1[k], gy1)
                inter = w * h
                iou = inter / (parea[k] + ga - inter)
                upd = iou > bval[k]
                nbval.append(jnp.where(upd, iou, bval[k]))
                nbcombo.append(jnp.where(upd, combo, bcombo[k]))
            return tuple(nbval), tuple(nbcombo)

        zero_f = jnp.zeros((LANES,), jnp.float32)
        combo0 = combo_v[0, :]
        init = ((zero_f,) * JB, (combo0,) * JB)
        bval, bcombo = lax.fori_loop(0, M, g_body, init)

        zero_i = jnp.zeros((LANES,), jnp.int32)
        for k in range(JB):
            ok = o + k * LANES
            fg = bval[k] >= IOU_THRESHOLD
            bidx = lax.shift_right_logical(bcombo[k], 7)
            cls = lax.bitwise_and(bcombo[k], zero_i + 127)
            vals_v[pl.ds(ok, LANES)] = bval[k]
            idxs_v[pl.ds(ok, LANES)] = bidx
            labs_v[pl.ds(ok, LANES)] = jnp.where(fg, zero_i + 1, zero_i)
            clso_v[pl.ds(ok, LANES)] = jnp.where(fg, cls, zero_i + NUM_CLASSES)
        return carry

    lax.fori_loop(0, VPW // JB, j_body, 0)

    pltpu.sync_copy(vals_v, vals_out.at[pl.ds(base, PPW)])
    pltpu.sync_copy(idxs_v, idxs_out.at[pl.ds(base, PPW)])
    pltpu.sync_copy(labs_v, labs_out.at[pl.ds(base, PPW)])
    pltpu.sync_copy(clso_v, cls_out.at[pl.ds(base, PPW)])


_roi = pl.kernel(
    _roi_body,
    out_type=(jax.ShapeDtypeStruct((N,), jnp.float32),
              jax.ShapeDtypeStruct((N,), jnp.int32),
              jax.ShapeDtypeStruct((N,), jnp.int32),
              jax.ShapeDtypeStruct((N,), jnp.int32)),
    mesh=plsc.VectorSubcoreMesh(core_axis_name="c", subcore_axis_name="s"),
    scratch_types=[
        pltpu.VMEM((PPW,), jnp.float32),
        pltpu.VMEM((PPW,), jnp.float32),
        pltpu.VMEM((PPW,), jnp.float32),
        pltpu.VMEM((PPW,), jnp.float32),
        pltpu.VMEM((5, M, LANES), jnp.float32),
        pltpu.VMEM((M, LANES), jnp.int32),
        pltpu.VMEM((PPW,), jnp.float32),
        pltpu.VMEM((PPW,), jnp.int32),
        pltpu.VMEM((PPW,), jnp.int32),
        pltpu.VMEM((PPW,), jnp.int32),
    ],
)


def kernel(proposal_boxes, gt_boxes, gt_classes):
    pb = jnp.transpose(proposal_boxes.astype(jnp.float32)).reshape(-1)
    gtt = jnp.transpose(gt_boxes.astype(jnp.float32))
    garea = ((gtt[2] - gtt[0]) * (gtt[3] - gtt[1]))[None]
    combo = (jnp.arange(M, dtype=jnp.int32) * 128
             + gt_classes.astype(jnp.int32))
    gt_all = jnp.concatenate([gtt, garea], axis=0)
    gt_rep = jnp.broadcast_to(gt_all[:, :, None], (5, M, LANES))
    combo_rep = jnp.broadcast_to(combo[:, None], (M, LANES))
    return _roi(pb, gt_rep, combo_rep)


# speedup vs baseline: 1.1945x; 1.1924x over previous
import jax
import jax.numpy as jnp
from jax import lax
from jax.experimental import pallas as pl
from jax.experimental.pallas import tpu as pltpu
from jax.experimental.pallas import tpu_sc as plsc

NUM_CLASSES = 80
IOU_THRESHOLD = 0.5
M = 100
N = 20000
LANES = 16
NW = 32
PPW = 384
SC_N = NW * PPW
TC_N = N - SC_N
VPW = PPW // LANES
JB = 4
TCB = 1024
TC_GRID = -(-TC_N // TCB)


def _sc_body(pb_hbm, gt_hbm, combo_hbm,
             vals_out, idxs_out, labs_out, cls_out,
             prop_v, gt_v, combo_v,
             vals_v, idxs_v, labs_v, clso_v):
    c = lax.axis_index("c")
    s = lax.axis_index("s")
    wid = s * 2 + c
    base = wid * PPW

    pltpu.sync_copy(pb_hbm.at[:, pl.ds(base, PPW)], prop_v)
    pltpu.sync_copy(gt_hbm, gt_v)
    pltpu.sync_copy(combo_hbm, combo_v)

    def j_body(j, carry):
        o = j * (LANES * JB)

        px1 = [prop_v[0, pl.ds(o + k * LANES, LANES)] for k in range(JB)]
        py1 = [prop_v[1, pl.ds(o + k * LANES, LANES)] for k in range(JB)]
        px2 = [prop_v[2, pl.ds(o + k * LANES, LANES)] for k in range(JB)]
        py2 = [prop_v[3, pl.ds(o + k * LANES, LANES)] for k in range(JB)]
        parea = [(px2[k] - px1[k]) * (py2[k] - py1[k]) for k in range(JB)]

        def g_body(g, st):
            bval, bcombo = st
            gx1 = gt_v[0, g, :]
            gy1 = gt_v[1, g, :]
            gx2 = gt_v[2, g, :]
            gy2 = gt_v[3, g, :]
            ga = gt_v[4, g, :]
            combo = combo_v[g, :]
            nbval = []
            nbcombo = []
            for k in range(JB):
                w = jnp.maximum(
                    jnp.minimum(px2[k], gx2) - jnp.maximum(px1[k], gx1), 0.0)
                h = jnp.minimum(py2[k], gy2) - jnp.maximum(py1[k], gy1)
                inter = w * h
                iou = inter / (parea[k] + ga - inter)
                upd = iou > bval[k]
                nbval.append(jnp.where(upd, iou, bval[k]))
                nbcombo.append(jnp.where(upd, combo, bcombo[k]))
            return tuple(nbval), tuple(nbcombo)

        zero_f = jnp.zeros((LANES,), jnp.float32)
        combo0 = combo_v[0, :]
        init = ((zero_f,) * JB, (combo0,) * JB)
        bval, bcombo = lax.fori_loop(0, M, g_body, init)

        zero_i = jnp.zeros((LANES,), jnp.int32)
        for k in range(JB):
            ok = o + k * LANES
            fg = bval[k] >= IOU_THRESHOLD
            bidx = lax.shift_right_logical(bcombo[k], 7)
            cls = lax.bitwise_and(bcombo[k], zero_i + 127)
            vals_v[pl.ds(ok, LANES)] = bval[k]
            idxs_v[pl.ds(ok, LANES)] = bidx
            labs_v[pl.ds(ok, LANES)] = jnp.where(fg, zero_i + 1, zero_i)
            clso_v[pl.ds(ok, LANES)] = jnp.where(fg, cls, zero_i + NUM_CLASSES)
        return carry

    lax.fori_loop(0, VPW // JB, j_body, 0)

    pltpu.sync_copy(vals_v, vals_out.at[pl.ds(base, PPW)])
    pltpu.sync_copy(idxs_v, idxs_out.at[pl.ds(base, PPW)])
    pltpu.sync_copy(labs_v, labs_out.at[pl.ds(base, PPW)])
    pltpu.sync_copy(clso_v, cls_out.at[pl.ds(base, PPW)])


_sc_match = pl.kernel(
    _sc_body,
    out_type=(jax.ShapeDtypeStruct((SC_N,), jnp.float32),
              jax.ShapeDtypeStruct((SC_N,), jnp.int32),
              jax.ShapeDtypeStruct((SC_N,), jnp.int32),
              jax.ShapeDtypeStruct((SC_N,), jnp.int32)),
    mesh=plsc.VectorSubcoreMesh(core_axis_name="c", subcore_axis_name="s"),
    scratch_types=[
        pltpu.VMEM((4, PPW), jnp.float32),
        pltpu.VMEM((5, M, LANES), jnp.float32),
        pltpu.VMEM((M, LANES), jnp.int32),
        pltpu.VMEM((PPW,), jnp.float32),
        pltpu.VMEM((PPW,), jnp.int32),
        pltpu.VMEM((PPW,), jnp.int32),
        pltpu.VMEM((PPW,), jnp.int32),
    ],
)


def _tc_body(gt_ref, combo_ref, pb_ref,
             vals_ref, idxs_ref, labs_ref, cls_ref):
    px1 = pb_ref[0, :]
    py1 = pb_ref[1, :]
    px2 = pb_ref[2, :]
    py2 = pb_ref[3, :]
    parea = (px2 - px1) * (py2 - py1)

    def g_body(g, st):
        bval, bcombo = st
        gx1 = gt_ref[0, g]
        gy1 = gt_ref[1, g]
        gx2 = gt_ref[2, g]
        gy2 = gt_ref[3, g]
        ga = gt_ref[4, g]
        combo = combo_ref[g]
        w = jnp.maximum(jnp.minimum(px2, gx2) - jnp.maximum(px1, gx1), 0.0)
        h = jnp.minimum(py2, gy2) - jnp.maximum(py1, gy1)
        inter = w * h
        iou = inter / (parea + ga - inter)
        upd = iou > bval
        bval = jnp.where(upd, iou, bval)
        bcombo = jnp.where(upd, combo, bcombo)
        return bval, bcombo

    init = (jnp.zeros((TCB,), jnp.float32),
            jnp.full((TCB,), combo_ref[0], jnp.int32))
    bval, bcombo = lax.fori_loop(0, M, g_body, init)

    fg = bval >= IOU_THRESHOLD
    bidx = lax.shift_right_logical(bcombo, 7)
    cls = lax.bitwise_and(bcombo, 127)
    vals_ref[...] = bval
    idxs_ref[...] = bidx
    labs_ref[...] = fg.astype(jnp.int32)
    cls_ref[...] = jnp.where(fg, cls, NUM_CLASSES)


_tc_match = pl.pallas_call(
    _tc_body,
    grid=(TC_GRID,),
    in_specs=[
        pl.BlockSpec(memory_space=pltpu.SMEM),
        pl.BlockSpec(memory_space=pltpu.SMEM),
        pl.BlockSpec((4, TCB), lambda i: (0, SC_N // TCB + i)),
    ],
    out_specs=[
        pl.BlockSpec((TCB,), lambda i: (i,)),
        pl.BlockSpec((TCB,), lambda i: (i,)),
        pl.BlockSpec((TCB,), lambda i: (i,)),
        pl.BlockSpec((TCB,), lambda i: (i,)),
    ],
    out_shape=(jax.ShapeDtypeStruct((TC_N,), jnp.float32),
               jax.ShapeDtypeStruct((TC_N,), jnp.int32),
               jax.ShapeDtypeStruct((TC_N,), jnp.int32),
               jax.ShapeDtypeStruct((TC_N,), jnp.int32)),
)


def kernel(proposal_boxes, gt_boxes, gt_classes):
    pb_t = jnp.transpose(proposal_boxes.astype(jnp.float32))
    gtt = jnp.transpose(gt_boxes.astype(jnp.float32))
    garea = ((gtt[2] - gtt[0]) * (gtt[3] - gtt[1]))[None]
    gt_all = jnp.concatenate([gtt, garea], axis=0)
    combo = (jnp.arange(M, dtype=jnp.int32) * 128
             + gt_classes.astype(jnp.int32))
    gt_rep = jnp.broadcast_to(gt_all[:, :, None], (5, M, LANES))
    combo_rep = jnp.broadcast_to(combo[:, None], (M, LANES))

    sc_out = _sc_match(pb_t, gt_rep, combo_rep)
    tc_out = _tc_match(gt_all, combo, pb_t)
    return tuple(jnp.concatenate([a, b]) for a, b in zip(sc_out, tc_out))


# speedup vs baseline: 1.2615x; 1.0561x over previous
import jax
import jax.numpy as jnp
from jax import lax
from jax.experimental import pallas as pl
from jax.experimental.pallas import tpu as pltpu
from jax.experimental.pallas import tpu_sc as plsc

NUM_CLASSES = 80
IOU_THRESHOLD = 0.5
M = 100
N = 20000
LANES = 16
NW = 32
PPW = 256
SC_N = NW * PPW
TC_N = N - SC_N
VPW = PPW // LANES
JB = 4
TCB = 4096
TC_GRID = -(-TC_N // TCB)


def _sc_body(pb_hbm, gt_hbm, combo_hbm,
             vals_out, idxs_out, labs_out, cls_out,
             prop_v, gt_v, combo_v,
             vals_v, idxs_v, labs_v, clso_v):
    c = lax.axis_index("c")
    s = lax.axis_index("s")
    wid = s * 2 + c
    base = wid * PPW

    pltpu.sync_copy(pb_hbm.at[:, pl.ds(base, PPW)], prop_v)
    pltpu.sync_copy(gt_hbm, gt_v)
    pltpu.sync_copy(combo_hbm, combo_v)

    def j_body(j, carry):
        o = j * (LANES * JB)

        px1 = [prop_v[0, pl.ds(o + k * LANES, LANES)] for k in range(JB)]
        py1 = [prop_v[1, pl.ds(o + k * LANES, LANES)] for k in range(JB)]
        px2 = [prop_v[2, pl.ds(o + k * LANES, LANES)] for k in range(JB)]
        py2 = [prop_v[3, pl.ds(o + k * LANES, LANES)] for k in range(JB)]
        parea = [(px2[k] - px1[k]) * (py2[k] - py1[k]) for k in range(JB)]

        def g_body(g, st):
            bval, bcombo = st
            gx1 = gt_v[0, g, :]
            gy1 = gt_v[1, g, :]
            gx2 = gt_v[2, g, :]
            gy2 = gt_v[3, g, :]
            ga = gt_v[4, g, :]
            combo = combo_v[g, :]
            nbval = []
            nbcombo = []
            for k in range(JB):
                w = jnp.maximum(
                    jnp.minimum(px2[k], gx2) - jnp.maximum(px1[k], gx1), 0.0)
                h = jnp.minimum(py2[k], gy2) - jnp.maximum(py1[k], gy1)
                inter = w * h
                iou = inter / (parea[k] + ga - inter)
                upd = iou > bval[k]
                nbval.append(jnp.where(upd, iou, bval[k]))
                nbcombo.append(jnp.where(upd, combo, bcombo[k]))
            return tuple(nbval), tuple(nbcombo)

        zero_f = jnp.zeros((LANES,), jnp.float32)
        combo0 = combo_v[0, :]
        init = ((zero_f,) * JB, (combo0,) * JB)
        bval, bcombo = lax.fori_loop(0, M, g_body, init)

        zero_i = jnp.zeros((LANES,), jnp.int32)
        for k in range(JB):
            ok = o + k * LANES
            fg = bval[k] >= IOU_THRESHOLD
            bidx = lax.shift_right_logical(bcombo[k], 7)
            cls = lax.bitwise_and(bcombo[k], zero_i + 127)
            vals_v[pl.ds(ok, LANES)] = bval[k]
            idxs_v[pl.ds(ok, LANES)] = bidx
            labs_v[pl.ds(ok, LANES)] = jnp.where(fg, zero_i + 1, zero_i)
            clso_v[pl.ds(ok, LANES)] = jnp.where(fg, cls, zero_i + NUM_CLASSES)
        return carry

    lax.fori_loop(0, VPW // JB, j_body, 0)

    pltpu.sync_copy(vals_v, vals_out.at[pl.ds(base, PPW)])
    pltpu.sync_copy(idxs_v, idxs_out.at[pl.ds(base, PPW)])
    pltpu.sync_copy(labs_v, labs_out.at[pl.ds(base, PPW)])
    pltpu.sync_copy(clso_v, cls_out.at[pl.ds(base, PPW)])


_sc_match = pl.kernel(
    _sc_body,
    out_type=(jax.ShapeDtypeStruct((SC_N,), jnp.float32),
              jax.ShapeDtypeStruct((SC_N,), jnp.int32),
              jax.ShapeDtypeStruct((SC_N,), jnp.int32),
              jax.ShapeDtypeStruct((SC_N,), jnp.int32)),
    mesh=plsc.VectorSubcoreMesh(core_axis_name="c", subcore_axis_name="s"),
    scratch_types=[
        pltpu.VMEM((4, PPW), jnp.float32),
        pltpu.VMEM((5, M, LANES), jnp.float32),
        pltpu.VMEM((M, LANES), jnp.int32),
        pltpu.VMEM((PPW,), jnp.float32),
        pltpu.VMEM((PPW,), jnp.int32),
        pltpu.VMEM((PPW,), jnp.int32),
        pltpu.VMEM((PPW,), jnp.int32),
    ],
)


def _tc_body(gt_ref, combo_ref, pb_ref,
             vals_ref, idxs_ref, labs_ref, cls_ref):
    px1 = pb_ref[0, :]
    py1 = pb_ref[1, :]
    px2 = pb_ref[2, :]
    py2 = pb_ref[3, :]
    parea = (px2 - px1) * (py2 - py1)

    def g_body(g, st):
        bval, bcombo = st
        gx1 = gt_ref[0, g]
        gy1 = gt_ref[1, g]
        gx2 = gt_ref[2, g]
        gy2 = gt_ref[3, g]
        ga = gt_ref[4, g]
        combo = combo_ref[g]
        w = jnp.maximum(jnp.minimum(px2, gx2) - jnp.maximum(px1, gx1), 0.0)
        h = jnp.minimum(py2, gy2) - jnp.maximum(py1, gy1)
        inter = w * h
        iou = inter / (parea + ga - inter)
        upd = iou > bval
        bval = jnp.where(upd, iou, bval)
        bcombo = jnp.where(upd, combo, bcombo)
        return bval, bcombo

    init = (jnp.zeros((TCB,), jnp.float32),
            jnp.full((TCB,), combo_ref[0], jnp.int32))
    bval, bcombo = lax.fori_loop(0, M, g_body, init)

    fg = bval >= IOU_THRESHOLD
    bidx = lax.shift_right_logical(bcombo, 7)
    cls = lax.bitwise_and(bcombo, 127)
    vals_ref[...] = bval
    idxs_ref[...] = bidx
    labs_ref[...] = fg.astype(jnp.int32)
    cls_ref[...] = jnp.where(fg, cls, NUM_CLASSES)


_tc_match = pl.pallas_call(
    _tc_body,
    grid=(TC_GRID,),
    in_specs=[
        pl.BlockSpec(memory_space=pltpu.SMEM),
        pl.BlockSpec(memory_space=pltpu.SMEM),
        pl.BlockSpec((4, TCB), lambda i: (0, SC_N // TCB + i)),
    ],
    out_specs=[
        pl.BlockSpec((TCB,), lambda i: (i,)),
        pl.BlockSpec((TCB,), lambda i: (i,)),
        pl.BlockSpec((TCB,), lambda i: (i,)),
        pl.BlockSpec((TCB,), lambda i: (i,)),
    ],
    out_shape=(jax.ShapeDtypeStruct((TC_N,), jnp.float32),
               jax.ShapeDtypeStruct((TC_N,), jnp.int32),
               jax.ShapeDtypeStruct((TC_N,), jnp.int32),
               jax.ShapeDtypeStruct((TC_N,), jnp.int32)),
)


def kernel(proposal_boxes, gt_boxes, gt_classes):
    pb_t = jnp.transpose(proposal_boxes.astype(jnp.float32))
    gtt = jnp.transpose(gt_boxes.astype(jnp.float32))
    garea = ((gtt[2] - gtt[0]) * (gtt[3] - gtt[1]))[None]
    gt_all = jnp.concatenate([gtt, garea], axis=0)
    combo = (jnp.arange(M, dtype=jnp.int32) * 128
             + gt_classes.astype(jnp.int32))
    gt_rep = jnp.broadcast_to(gt_all[:, :, None], (5, M, LANES))
    combo_rep = jnp.broadcast_to(combo[:, None], (M, LANES))

    sc_out = _sc_match(pb_t, gt_rep, combo_rep)
    tc_out = _tc_match(gt_all, combo, pb_t)
    return tuple(jnp.concatenate([a, b]) for a, b in zip(sc_out, tc_out))


# speedup vs baseline: 1.2764x; 1.0118x over previous
import jax
import jax.numpy as jnp
from jax import lax
from jax.experimental import pallas as pl
from jax.experimental.pallas import tpu as pltpu
from jax.experimental.pallas import tpu_sc as plsc

NUM_CLASSES = 80
IOU_THRESHOLD = 0.5
M = 100
N = 20000
LANES = 16
NW = 32
PPW = 256
SC_N = NW * PPW
TC_N = N - SC_N
VPW = PPW // LANES
JB = 4
TCB = 4096
TC_GRID = -(-TC_N // TCB)


def _sc_body(pb_hbm, gt_hbm, combo_hbm,
             vals_out, idxs_out, labs_out, cls_out,
             prop_v, gt_v, combo_v,
             vals_v, idxs_v, labs_v, clso_v, dsem):
    c = lax.axis_index("c")
    s = lax.axis_index("s")
    wid = s * 2 + c
    base = wid * PPW

    cp1 = pltpu.async_copy(pb_hbm.at[:, pl.ds(base, PPW)], prop_v, dsem)
    cp2 = pltpu.async_copy(gt_hbm, gt_v, dsem)
    cp3 = pltpu.async_copy(combo_hbm, combo_v, dsem)
    cp1.wait()
    cp2.wait()
    cp3.wait()

    def j_body(j, carry):
        o = j * (LANES * JB)

        px1 = [prop_v[0, pl.ds(o + k * LANES, LANES)] for k in range(JB)]
        py1 = [prop_v[1, pl.ds(o + k * LANES, LANES)] for k in range(JB)]
        px2 = [prop_v[2, pl.ds(o + k * LANES, LANES)] for k in range(JB)]
        py2 = [prop_v[3, pl.ds(o + k * LANES, LANES)] for k in range(JB)]
        parea = [(px2[k] - px1[k]) * (py2[k] - py1[k]) for k in range(JB)]

        def g_body(g, st):
            bval, bcombo = st
            gx1 = gt_v[0, g, :]
            gy1 = gt_v[1, g, :]
            gx2 = gt_v[2, g, :]
            gy2 = gt_v[3, g, :]
            ga = gt_v[4, g, :]
            combo = combo_v[g, :]
            nbval = []
            nbcombo = []
            for k in range(JB):
                w = jnp.maximum(
                    jnp.minimum(px2[k], gx2) - jnp.maximum(px1[k], gx1), 0.0)
                h = jnp.minimum(py2[k], gy2) - jnp.maximum(py1[k], gy1)
                inter = w * h
                iou = inter / (parea[k] + ga - inter)
                upd = iou > bval[k]
                nbval.append(jnp.where(upd, iou, bval[k]))
                nbcombo.append(jnp.where(upd, combo, bcombo[k]))
            return tuple(nbval), tuple(nbcombo)

        zero_f = jnp.zeros((LANES,), jnp.float32)
        combo0 = combo_v[0, :]
        init = ((zero_f,) * JB, (combo0,) * JB)
        bval, bcombo = lax.fori_loop(0, M, g_body, init)

        zero_i = jnp.zeros((LANES,), jnp.int32)
        for k in range(JB):
            ok = o + k * LANES
            fg = bval[k] >= IOU_THRESHOLD
            bidx = lax.shift_right_logical(bcombo[k], 7)
            cls = lax.bitwise_and(bcombo[k], zero_i + 127)
            vals_v[pl.ds(ok, LANES)] = bval[k]
            idxs_v[pl.ds(ok, LANES)] = bidx
            labs_v[pl.ds(ok, LANES)] = jnp.where(fg, zero_i + 1, zero_i)
            clso_v[pl.ds(ok, LANES)] = jnp.where(fg, cls, zero_i + NUM_CLASSES)
        return carry

    lax.fori_loop(0, VPW // JB, j_body, 0)

    pltpu.sync_copy(vals_v, vals_out.at[pl.ds(base, PPW)])
    pltpu.sync_copy(idxs_v, idxs_out.at[pl.ds(base, PPW)])
    pltpu.sync_copy(labs_v, labs_out.at[pl.ds(base, PPW)])
    pltpu.sync_copy(clso_v, cls_out.at[pl.ds(base, PPW)])


_sc_match = pl.kernel(
    _sc_body,
    out_type=(jax.ShapeDtypeStruct((SC_N,), jnp.float32),
              jax.ShapeDtypeStruct((SC_N,), jnp.int32),
              jax.ShapeDtypeStruct((SC_N,), jnp.int32),
              jax.ShapeDtypeStruct((SC_N,), jnp.int32)),
    mesh=plsc.VectorSubcoreMesh(core_axis_name="c", subcore_axis_name="s"),
    scratch_types=[
        pltpu.VMEM((4, PPW), jnp.float32),
        pltpu.VMEM((5, M, LANES), jnp.float32),
        pltpu.VMEM((M, LANES), jnp.int32),
        pltpu.VMEM((PPW,), jnp.float32),
        pltpu.VMEM((PPW,), jnp.int32),
        pltpu.VMEM((PPW,), jnp.int32),
        pltpu.VMEM((PPW,), jnp.int32),
        pltpu.SemaphoreType.DMA,
    ],
)


def _tc_body(gt_ref, combo_ref, pb_ref,
             vals_ref, idxs_ref, labs_ref, cls_ref):
    px1 = pb_ref[0, :]
    py1 = pb_ref[1, :]
    px2 = pb_ref[2, :]
    py2 = pb_ref[3, :]
    parea = (px2 - px1) * (py2 - py1)

    def g_body(g, st):
        bval, bcombo = st
        gx1 = gt_ref[0, g]
        gy1 = gt_ref[1, g]
        gx2 = gt_ref[2, g]
        gy2 = gt_ref[3, g]
        ga = gt_ref[4, g]
        combo = combo_ref[g]
        w = jnp.maximum(jnp.minimum(px2, gx2) - jnp.maximum(px1, gx1), 0.0)
        h = jnp.minimum(py2, gy2) - jnp.maximum(py1, gy1)
        inter = w * h
        iou = inter / (parea + ga - inter)
        upd = iou > bval
        bval = jnp.where(upd, iou, bval)
        bcombo = jnp.where(upd, combo, bcombo)
        return bval, bcombo

    init = (jnp.zeros((TCB,), jnp.float32),
            jnp.full((TCB,), combo_ref[0], jnp.int32))
    bval, bcombo = lax.fori_loop(0, M, g_body, init)

    fg = bval >= IOU_THRESHOLD
    bidx = lax.shift_right_logical(bcombo, 7)
    cls = lax.bitwise_and(bcombo, 127)
    vals_ref[...] = bval
    idxs_ref[...] = bidx
    labs_ref[...] = fg.astype(jnp.int32)
    cls_ref[...] = jnp.where(fg, cls, NUM_CLASSES)


_tc_match = pl.pallas_call(
    _tc_body,
    grid=(TC_GRID,),
    in_specs=[
        pl.BlockSpec(memory_space=pltpu.SMEM),
        pl.BlockSpec(memory_space=pltpu.SMEM),
        pl.BlockSpec((4, TCB), lambda i: (0, SC_N // TCB + i)),
    ],
    out_specs=[
        pl.BlockSpec((TCB,), lambda i: (i,)),
        pl.BlockSpec((TCB,), lambda i: (i,)),
        pl.BlockSpec((TCB,), lambda i: (i,)),
        pl.BlockSpec((TCB,), lambda i: (i,)),
    ],
    out_shape=(jax.ShapeDtypeStruct((TC_N,), jnp.float32),
               jax.ShapeDtypeStruct((TC_N,), jnp.int32),
               jax.ShapeDtypeStruct((TC_N,), jnp.int32),
               jax.ShapeDtypeStruct((TC_N,), jnp.int32)),
)


def kernel(proposal_boxes, gt_boxes, gt_classes):
    pb_t = jnp.transpose(proposal_boxes.astype(jnp.float32))
    gtt = jnp.transpose(gt_boxes.astype(jnp.float32))
    garea = ((gtt[2] - gtt[0]) * (gtt[3] - gtt[1]))[None]
    gt_all = jnp.concatenate([gtt, garea], axis=0)
    combo = (jnp.arange(M, dtype=jnp.int32) * 128
             + gt_classes.astype(jnp.int32))
    gt_rep = jnp.broadcast_to(gt_all[:, :, None], (5, M, LANES))
    combo_rep = jnp.broadcast_to(combo[:, None], (M, LANES))

    sc_out = _sc_match(pb_t, gt_rep, combo_rep)
    tc_out = _tc_match(gt_all, combo, pb_t)
    return tuple(jnp.concatenate([a, b]) for a, b in zip(sc_out, tc_out))


# speedup vs baseline: 1.3774x; 1.0791x over previous
import jax
import jax.numpy as jnp
from jax import lax
from jax.experimental import pallas as pl
from jax.experimental.pallas import tpu as pltpu
from jax.experimental.pallas import tpu_sc as plsc

NUM_CLASSES = 80
IOU_THRESHOLD = 0.5
M = 100
N = 20000
LANES = 16
NW = 32
PPW = 128
SC_N = NW * PPW
TC_N = N - SC_N
VPW = PPW // LANES
JB = 4
TCB = 4096
TC_GRID = -(-TC_N // TCB)


def _sc_body(pb_hbm, gt_hbm, combo_hbm,
             vals_out, idxs_out, labs_out, cls_out,
             prop_v, gt_v, combo_v,
             vals_v, idxs_v, labs_v, clso_v, dsem):
    c = lax.axis_index("c")
    s = lax.axis_index("s")
    wid = s * 2 + c
    base = wid * PPW

    cp1 = pltpu.async_copy(pb_hbm.at[:, pl.ds(base, PPW)], prop_v, dsem)
    cp2 = pltpu.async_copy(gt_hbm, gt_v, dsem)
    cp3 = pltpu.async_copy(combo_hbm, combo_v, dsem)
    cp1.wait()
    cp2.wait()
    cp3.wait()

    def j_body(j, carry):
        o = j * (LANES * JB)

        px1 = [prop_v[0, pl.ds(o + k * LANES, LANES)] for k in range(JB)]
        py1 = [prop_v[1, pl.ds(o + k * LANES, LANES)] for k in range(JB)]
        px2 = [prop_v[2, pl.ds(o + k * LANES, LANES)] for k in range(JB)]
        py2 = [prop_v[3, pl.ds(o + k * LANES, LANES)] for k in range(JB)]
        parea = [(px2[k] - px1[k]) * (py2[k] - py1[k]) for k in range(JB)]

        def g_body(g, st):
            bval, bcombo = st
            gx1 = gt_v[0, g, :]
            gy1 = gt_v[1, g, :]
            gx2 = gt_v[2, g, :]
            gy2 = gt_v[3, g, :]
            ga = gt_v[4, g, :]
            combo = combo_v[g, :]
            nbval = []
            nbcombo = []
            for k in range(JB):
                w = jnp.maximum(
                    jnp.minimum(px2[k], gx2) - jnp.maximum(px1[k], gx1), 0.0)
                h = jnp.minimum(py2[k], gy2) - jnp.maximum(py1[k], gy1)
                inter = w * h
                iou = inter / (parea[k] + ga - inter)
                upd = iou > bval[k]
                nbval.append(jnp.where(upd, iou, bval[k]))
                nbcombo.append(jnp.where(upd, combo, bcombo[k]))
            return tuple(nbval), tuple(nbcombo)

        zero_f = jnp.zeros((LANES,), jnp.float32)
        combo0 = combo_v[0, :]
        init = ((zero_f,) * JB, (combo0,) * JB)
        bval, bcombo = lax.fori_loop(0, M, g_body, init)

        zero_i = jnp.zeros((LANES,), jnp.int32)
        for k in range(JB):
            ok = o + k * LANES
            fg = bval[k] >= IOU_THRESHOLD
            bidx = lax.shift_right_logical(bcombo[k], 7)
            cls = lax.bitwise_and(bcombo[k], zero_i + 127)
            vals_v[pl.ds(ok, LANES)] = bval[k]
            idxs_v[pl.ds(ok, LANES)] = bidx
            labs_v[pl.ds(ok, LANES)] = jnp.where(fg, zero_i + 1, zero_i)
            clso_v[pl.ds(ok, LANES)] = jnp.where(fg, cls, zero_i + NUM_CLASSES)
        return carry

    lax.fori_loop(0, VPW // JB, j_body, 0)

    co1 = pltpu.async_copy(vals_v, vals_out.at[pl.ds(base, PPW)], dsem)
    co2 = pltpu.async_copy(idxs_v, idxs_out.at[pl.ds(base, PPW)], dsem)
    co3 = pltpu.async_copy(labs_v, labs_out.at[pl.ds(base, PPW)], dsem)
    co4 = pltpu.async_copy(clso_v, cls_out.at[pl.ds(base, PPW)], dsem)
    co1.wait()
    co2.wait()
    co3.wait()
    co4.wait()


_sc_match = pl.kernel(
    _sc_body,
    out_type=(jax.ShapeDtypeStruct((SC_N,), jnp.float32),
              jax.ShapeDtypeStruct((SC_N,), jnp.int32),
              jax.ShapeDtypeStruct((SC_N,), jnp.int32),
              jax.ShapeDtypeStruct((SC_N,), jnp.int32)),
    mesh=plsc.VectorSubcoreMesh(core_axis_name="c", subcore_axis_name="s"),
    scratch_types=[
        pltpu.VMEM((4, PPW), jnp.float32),
        pltpu.VMEM((5, M, LANES), jnp.float32),
        pltpu.VMEM((M, LANES), jnp.int32),
        pltpu.VMEM((PPW,), jnp.float32),
        pltpu.VMEM((PPW,), jnp.int32),
        pltpu.VMEM((PPW,), jnp.int32),
        pltpu.VMEM((PPW,), jnp.int32),
        pltpu.SemaphoreType.DMA,
    ],
)


def _tc_body(gt_ref, combo_ref, pb_ref,
             vals_ref, idxs_ref, labs_ref, cls_ref):
    px1 = pb_ref[0, :]
    py1 = pb_ref[1, :]
    px2 = pb_ref[2, :]
    py2 = pb_ref[3, :]
    parea = (px2 - px1) * (py2 - py1)

    def g_body(g, st):
        bval, bcombo = st
        gx1 = gt_ref[0, g]
        gy1 = gt_ref[1, g]
        gx2 = gt_ref[2, g]
        gy2 = gt_ref[3, g]
        ga = gt_ref[4, g]
        combo = combo_ref[g]
        w = jnp.maximum(jnp.minimum(px2, gx2) - jnp.maximum(px1, gx1), 0.0)
        h = jnp.minimum(py2, gy2) - jnp.maximum(py1, gy1)
        inter = w * h
        iou = inter / (parea + ga - inter)
        upd = iou > bval
        bval = jnp.where(upd, iou, bval)
        bcombo = jnp.where(upd, combo, bcombo)
        return bval, bcombo

    init = (jnp.zeros((TCB,), jnp.float32),
            jnp.full((TCB,), combo_ref[0], jnp.int32))
    bval, bcombo = lax.fori_loop(0, M, g_body, init)

    fg = bval >= IOU_THRESHOLD
    bidx = lax.shift_right_logical(bcombo, 7)
    cls = lax.bitwise_and(bcombo, 127)
    vals_ref[...] = bval
    idxs_ref[...] = bidx
    labs_ref[...] = fg.astype(jnp.int32)
    cls_ref[...] = jnp.where(fg, cls, NUM_CLASSES)


_tc_match = pl.pallas_call(
    _tc_body,
    grid=(TC_GRID,),
    in_specs=[
        pl.BlockSpec(memory_space=pltpu.SMEM),
        pl.BlockSpec(memory_space=pltpu.SMEM),
        pl.BlockSpec((4, TCB), lambda i: (0, SC_N // TCB + i)),
    ],
    out_specs=[
        pl.BlockSpec((TCB,), lambda i: (i,)),
        pl.BlockSpec((TCB,), lambda i: (i,)),
        pl.BlockSpec((TCB,), lambda i: (i,)),
        pl.BlockSpec((TCB,), lambda i: (i,)),
    ],
    out_shape=(jax.ShapeDtypeStruct((TC_N,), jnp.float32),
               jax.ShapeDtypeStruct((TC_N,), jnp.int32),
               jax.ShapeDtypeStruct((TC_N,), jnp.int32),
               jax.ShapeDtypeStruct((TC_N,), jnp.int32)),
)


def kernel(proposal_boxes, gt_boxes, gt_classes):
    pb_t = jnp.transpose(proposal_boxes.astype(jnp.float32))
    gtt = jnp.transpose(gt_boxes.astype(jnp.float32))
    garea = ((gtt[2] - gtt[0]) * (gtt[3] - gtt[1]))[None]
    gt_all = jnp.concatenate([gtt, garea], axis=0)
    combo = (jnp.arange(M, dtype=jnp.int32) * 128
             + gt_classes.astype(jnp.int32))
    gt_rep = jnp.broadcast_to(gt_all[:, :, None], (5, M, LANES))
    combo_rep = jnp.broadcast_to(combo[:, None], (M, LANES))

    sc_out = _sc_match(pb_t, gt_rep, combo_rep)
    tc_out = _tc_match(gt_all, combo, pb_t)
    return tuple(jnp.concatenate([a, b]) for a, b in zip(sc_out, tc_out))
